# baseline (speedup 1.0000x reference)
import functools

import numpy as np
import jax
import jax.numpy as jnp
from jax import lax
from jax.experimental import pallas as pl
from jax.experimental.pallas import tpu as pltpu, tpu_sc as plsc

N_INST = 8192
N_NET = 4096
EMB = 64
NUM_VN = 16

BR = 256
BN = 256
NGRP = 128
NW = 32
ROWS_W = N_INST // NW
RC = 64
CAP = 4096
GC = 128


def _lrelu(v):
    return jnp.where(v >= 0, v, 0.1 * v)


def _dot(a, b):
    return jnp.dot(a, b, preferred_element_type=jnp.float32)


def _pack_matrix():
    j = np.arange(N_NET)
    grp, sub = j // 32, j % 32
    col = np.where(sub < 16, grp, NGRP + grp)
    P = np.zeros((N_NET, 2 * NGRP), np.float32)
    P[j, col] = 2.0 ** (sub % 16)
    return jnp.asarray(P)


def _enc_inst_body(x_ref, w1_ref, b1_ref, w2_ref, b2_ref, o_ref):
    h = _lrelu(_dot(x_ref[...], w1_ref[...].T) + b1_ref[...])
    o_ref[...] = _lrelu(_dot(h, w2_ref[...].T) + b2_ref[...])


def _enc_net_body(x_ref, w1_ref, b1_ref, w2_ref, b2_ref,
                  p1_ref, pb1_ref, p2_ref, pb2_ref, hn_ref, na_ref):
    h = _lrelu(_dot(x_ref[...], w1_ref[...].T) + b1_ref[...])
    hn = _lrelu(_dot(h, w2_ref[...].T) + b2_ref[...])
    hn_ref[...] = hn
    t = jax.nn.relu(_dot(hn, p1_ref[...].T) + pb1_ref[...])
    na_ref[...] = _dot(t, p2_ref[...].T) + pb2_ref[...]


def _tail_math(h_in, hd, hs0, psi1, psib1, psi2, psib2,
               m1, mb1, m2, mb2, g, be):
    hs = _dot(jax.nn.relu(_dot(hs0, psi1.T) + psib1), psi2.T) + psib2
    hc = jnp.concatenate([h_in, hd, hs], axis=1)
    hm = jax.nn.relu(_dot(hc, m1.T) + mb1)
    ho = _dot(hm, m2.T) + mb2
    mu = jnp.mean(ho, axis=-1, keepdims=True)
    var = jnp.mean((ho - mu) ** 2, axis=-1, keepdims=True)
    ln = (ho - mu) / jnp.sqrt(var + 1e-5) * g + be
    return ho, _lrelu(ln) + h_in


def _layer0_body(drive_ref, sink_ref, na_ref, hb_ref, oh_ref, vnt_ref, P_ref,
                 psi1_ref, psib1_ref, psi2_ref, psib2_ref,
                 m1_ref, mb1_ref, m2_ref, mb2_ref, g_ref, be_ref,
                 q1_ref, qb1_ref, q2_ref, qb2_ref,
                 ho_ref, hin_ref, hpre_ref, bmd_ref, bms_ref,
                 pool_ref, vn_next_ref):
    i = pl.program_id(0)
    h_in = hb_ref[...] + _dot(oh_ref[...], vnt_ref[...])
    hin_ref[...] = h_in
    drive = drive_ref[...]
    sink = sink_ref[...]
    hd = _dot(drive, na_ref[...])
    hs0 = _dot(sink, na_ref[...])
    P = P_ref[...]
    for adj, bm_ref in ((drive, bmd_ref), (sink, bms_ref)):
        bits = _dot(adj, P)
        lo = bits[:, :NGRP].astype(jnp.int32)
        hi = bits[:, NGRP:].astype(jnp.int32)
        bm_ref[...] = lo + hi * 65536

    hpre, hout = _tail_math(
        h_in, hd, hs0, psi1_ref[...], psib1_ref[...], psi2_ref[...],
        psib2_ref[...], m1_ref[...], mb1_ref[...], m2_ref[...], mb2_ref[...],
        g_ref[...], be_ref[...])
    hpre_ref[...] = hpre
    ho_ref[...] = hout

    ones = jnp.ones((h_in.shape[0], EMB), jnp.float32)
    hp = jnp.concatenate([h_in, ones], axis=1)
    contrib = _dot(oh_ref[...].T, hp)

    @pl.when(i == 0)
    def _():
        pool_ref[...] = jnp.zeros_like(pool_ref)

    pool_ref[...] += contrib

    @pl.when(i == pl.num_programs(0) - 1)
    def _():
        pool = pool_ref[...]
        counts = jnp.maximum(pool[:, EMB:EMB + 1], 1.0)
        vn_in = pool[:, :EMB] / counts + vnt_ref[...]
        t = _lrelu(_dot(vn_in, q1_ref[...].T) + qb1_ref[...])
        vn_next_ref[...] = _lrelu(_dot(t, q2_ref[...].T) + qb2_ref[...])


def _tail1_body(hd_ref, hs0_ref, hb_ref, oh_ref, vnt_ref,
                psi1_ref, psib1_ref, psi2_ref, psib2_ref,
                m1_ref, mb1_ref, m2_ref, mb2_ref, g_ref, be_ref,
                ho_ref, hin_ref):
    h_in = hb_ref[...] + _dot(oh_ref[...], vnt_ref[...])
    hin_ref[...] = h_in
    _, hout = _tail_math(
        h_in, hd_ref[...], hs0_ref[...], psi1_ref[...], psib1_ref[...],
        psi2_ref[...], psib2_ref[...], m1_ref[...], mb1_ref[...],
        m2_ref[...], mb2_ref[...], g_ref[...], be_ref[...])
    ho_ref[...] = hout


def _net_body(adj_ref, ho_ref, hn_ref, p1_ref, pb1_ref, p2_ref, pb2_ref,
              hn1_ref, na1_ref):
    hn1 = _dot(adj_ref[...], ho_ref[...]) + hn_ref[...]
    hn1_ref[...] = hn1
    t = jax.nn.relu(_dot(hn1, p1_ref[...].T) + pb1_ref[...])
    na1 = _dot(t, p2_ref[...].T) + pb2_ref[...]
    na1_ref[...] = jnp.concatenate(
        [na1, jnp.zeros_like(na1)], axis=1)


SEG = CAP // 16
RES = 32
NCH_S = 20


def _ctz(w):
    lsb = w & (-w)
    fb = lax.bitcast_convert_type(lsb.astype(jnp.float32), jnp.int32)
    return (lax.shift_right_logical(fb, 23) & 255) - 127


def _sc_spmv_body(bmd_hbm, bms_hbm, na_hbm, hd_hbm, hs_hbm,
                  bmchunk, colA, rowA, colB, rowB, resw, respos, gbuf, acc,
                  ptrs_ref, sem, ph=9):
    wid = lax.axis_index("s") * 2 + lax.axis_index("c")
    word0 = wid * ROWS_W * NGRP
    iota = lax.iota(jnp.int32, 16)
    segbase = iota * SEG

    def clear(ref, n, dt):
        def zb(k, _):
            ref[pl.ds(k * 16, 16)] = jnp.zeros((16,), dt)
            return 0
        lax.fori_loop(0, n // 16, zb, 0)

    def clear_acc():
        def zb(k, _):
            acc[k >> 2, pl.ds((k & 3) * 16, 16)] = jnp.zeros((16,),
                                                            jnp.float32)
            return 0
        lax.fori_loop(0, ROWS_W * 4, zb, 0)

    def do_matrix(bm_hbm, out_hbm):
        clear(colA, CAP, jnp.int32)
        clear(colB, CAP, jnp.int32)
        clear(resw, RES * 16, jnp.int32)
        clear(ptrs_ref, 32, jnp.int32)
        clear_acc()

        def scan_chunk(c):
            pltpu.sync_copy(
                bm_hbm.at[pl.ds(word0 + c * (RC * NGRP), RC * NGRP)], bmchunk)

            def scan_vreg(v, _):
                ptrs = ptrs_ref[pl.ds(0, 16)]
                rptrs = ptrs_ref[pl.ds(16, 16)]
                w = bmchunk[pl.ds(v * 16, 16)]
                m = w != 0
                mi = m.astype(jnp.int32)
                pos = c * (RC * NGRP) + v * 16 + iota
                row = lax.shift_right_logical(pos, 7)
                colb = (pos & (NGRP - 1)) * 32
                col = colb + _ctz(w)
                plsc.store_scatter(colA, [segbase + ptrs], col, mask=m)
                plsc.store_scatter(rowA, [segbase + ptrs], row, mask=m)
                w2 = w & (w - 1)
                m2 = w2 != 0
                plsc.store_scatter(resw, [rptrs * 16 + iota], w2, mask=m2)
                plsc.store_scatter(respos, [rptrs * 16 + iota], pos, mask=m2)
                ptrs_ref[pl.ds(0, 16)] = ptrs + mi
                ptrs_ref[pl.ds(16, 16)] = rptrs + m2.astype(jnp.int32)
                return 0

            lax.fori_loop(0, RC * NGRP // 16, scan_vreg, 0)

        if ph >= 1:
            for c in range(ROWS_W // RC):
                scan_chunk(c)

        for e in range(RES if ph >= 2 else 0):
            w = resw[pl.ds(e * 16, 16)]
            pos = respos[pl.ds(e * 16, 16)]
            row = lax.shift_right_logical(pos, 7)
            colb = (pos & (NGRP - 1)) * 32
            for _ in range(8):
                ptrs = ptrs_ref[pl.ds(0, 16)]
                m = w != 0
                col = colb + _ctz(w)
                plsc.store_scatter(colA, [segbase + ptrs], col, mask=m)
                plsc.store_scatter(rowA, [segbase + ptrs], row, mask=m)
                ptrs_ref[pl.ds(0, 16)] = ptrs + m.astype(jnp.int32)
                w = w & (w - 1)
        ptrs = ptrs_ref[pl.ds(0, 16)]

        def compact(l, gptr):
            n_l = ptrs[l]

            def cp(k, _):
                cv = colA[pl.ds(l * SEG + k * 16, 16)]
                rv = rowA[pl.ds(l * SEG + k * 16, 16)]
                tgt = gptr + k * 16 + iota
                plsc.store_scatter(colB, [tgt], cv)
                plsc.store_scatter(rowB, [tgt], rv)
                return 0

            lax.fori_loop(0, (n_l + 15) // 16, cp, 0)
            return gptr + n_l

        nnz = 0
        if ph >= 3:
            for l in range(16):
                nnz = compact(l, nnz)

        def accum_chunk(g, b):
            lim = jnp.minimum(GC, nnz - g * GC)

            def accum(e, _):
                r = rowB[pl.ds(g * GC + e, 16)][0]
                for q in range(4):
                    plsc.addupdate(acc.at[r, pl.ds(q * 16, 16)],
                                   gbuf[b, e, pl.ds(q * 16, 16)])
                return 0

            lax.fori_loop(0, lim, accum, 0)

        sems = (sem, sem2)
        h = [None, None]
        h[0] = pltpu.async_copy(
            na_hbm.at[colB.at[pl.ds(0, GC)]], gbuf.at[0], sems[0])
        for g in range(NCH_S):
            b = g & 1
            if g + 1 < NCH_S:
                h[1 - b] = pltpu.async_copy(
                    na_hbm.at[colB.at[pl.ds((g + 1) * GC, GC)]],
                    gbuf.at[1 - b], sems[1 - b])
            h[b].wait()
            accum_chunk(g, b)
        pltpu.sync_copy(acc, out_hbm.at[pl.ds(wid * ROWS_W, ROWS_W), :])

    do_matrix(bmd_hbm, hd_hbm)
    do_matrix(bms_hbm, hs_hbm)


def _sc_spmv(bmd, bms, na1p, ph=9):
    mesh = plsc.VectorSubcoreMesh(core_axis_name="c", subcore_axis_name="s")
    f = pl.kernel(
        functools.partial(_sc_spmv_body, ph=ph),
        mesh=mesh,
        compiler_params=pltpu.CompilerParams(needs_layout_passes=False),
        out_type=[jax.ShapeDtypeStruct((N_INST, EMB), jnp.float32),
                  jax.ShapeDtypeStruct((N_INST, EMB), jnp.float32)],
        scratch_types=[
            pltpu.VMEM((RC * NGRP,), jnp.int32),
            pltpu.VMEM((CAP,), jnp.int32),
            pltpu.VMEM((CAP,), jnp.int32),
            pltpu.VMEM((CAP + 16,), jnp.int32),
            pltpu.VMEM((CAP + 16,), jnp.int32),
            pltpu.VMEM((RES * 16,), jnp.int32),
            pltpu.VMEM((RES * 16,), jnp.int32),
            pltpu.VMEM((GC, 2 * EMB), jnp.float32),
            pltpu.VMEM((ROWS_W, EMB), jnp.float32),
            pltpu.VMEM((32,), jnp.int32),
            pltpu.SemaphoreType.DMA,
        ],
    )
    return f(bmd, bms, na1p)


def _full(shape):
    return pl.BlockSpec(shape, lambda i: tuple(0 for _ in shape))


def _rows(bs, width):
    return pl.BlockSpec((bs, width), lambda i: (i, 0))


def kernel(x, x_net, net_inst_adj, inst_net_adj_v_drive, inst_net_adj_v_sink,
           batch, num_vn, params):
    p = params
    r2 = lambda a: a.reshape(1, -1)
    oh = (batch[:, None] == jnp.arange(NUM_VN, dtype=batch.dtype)[None, :]
          ).astype(jnp.float32)
    vn0 = jnp.tile(p["vn_emb"], (NUM_VN, 1)) + 0.0 * num_vn
    Pm = _pack_matrix()

    h0 = pl.pallas_call(
        _enc_inst_body,
        grid=(8,),
        in_specs=[_rows(N_INST // 8, x.shape[1]),
                  _full(p["enc_W1"].shape), _full((1, 2 * EMB)),
                  _full(p["enc_W2"].shape), _full((1, EMB))],
        out_specs=_rows(N_INST // 8, EMB),
        out_shape=jax.ShapeDtypeStruct((N_INST, EMB), jnp.float32),
    )(x, p["enc_W1"], r2(p["enc_b1"]), p["enc_W2"], r2(p["enc_b2"]))

    L0, L1 = p["layers"][0], p["layers"][1]
    hn0, na0 = pl.pallas_call(
        _enc_net_body,
        grid=(4,),
        in_specs=[_rows(N_NET // 4, x_net.shape[1]),
                  _full(p["encnet_W1"].shape), _full((1, EMB)),
                  _full(p["encnet_W2"].shape), _full((1, EMB)),
                  _full(L0["phi_W1"].shape), _full((1, EMB)),
                  _full(L0["phi_W2"].shape), _full((1, EMB))],
        out_specs=[_rows(N_NET // 4, EMB), _rows(N_NET // 4, EMB)],
        out_shape=[jax.ShapeDtypeStruct((N_NET, EMB), jnp.float32),
                   jax.ShapeDtypeStruct((N_NET, EMB), jnp.float32)],
    )(x_net, p["encnet_W1"], r2(p["encnet_b1"]), p["encnet_W2"],
      r2(p["encnet_b2"]), L0["phi_W1"], r2(L0["phi_b1"]),
      L0["phi_W2"], r2(L0["phi_b2"]))

    q0 = p["vn_mlp"][0]
    h_out0, h_in0, h_pre0, bmd, bms, _, vn1 = pl.pallas_call(
        _layer0_body,
        grid=(N_INST // BR,),
        in_specs=[_rows(BR, N_NET), _rows(BR, N_NET), _full((N_NET, EMB)),
                  _rows(BR, EMB), _rows(BR, NUM_VN), _full((NUM_VN, EMB)),
                  _full((N_NET, 2 * NGRP)),
                  _full(L0["psi_W1"].shape), _full((1, EMB)),
                  _full(L0["psi_W2"].shape), _full((1, EMB)),
                  _full(L0["mlp_W1"].shape), _full((1, 3 * EMB)),
                  _full(L0["mlp_W2"].shape), _full((1, EMB)),
                  _full((1, EMB)), _full((1, EMB)),
                  _full(q0["W1"].shape), _full((1, 2 * EMB)),
                  _full(q0["W2"].shape), _full((1, EMB))],
        out_specs=[_rows(BR, EMB), _rows(BR, EMB), _rows(BR, EMB),
                   _rows(BR, NGRP), _rows(BR, NGRP),
                   _full((NUM_VN, 2 * EMB)), _full((NUM_VN, EMB))],
        out_shape=[jax.ShapeDtypeStruct((N_INST, EMB), jnp.float32),
                   jax.ShapeDtypeStruct((N_INST, EMB), jnp.float32),
                   jax.ShapeDtypeStruct((N_INST, EMB), jnp.float32),
                   jax.ShapeDtypeStruct((N_INST, NGRP), jnp.int32),
                   jax.ShapeDtypeStruct((N_INST, NGRP), jnp.int32),
                   jax.ShapeDtypeStruct((NUM_VN, 2 * EMB), jnp.float32),
                   jax.ShapeDtypeStruct((NUM_VN, EMB), jnp.float32)],
    )(inst_net_adj_v_drive, inst_net_adj_v_sink, na0, h0, oh, vn0, Pm,
      L0["psi_W1"], r2(L0["psi_b1"]), L0["psi_W2"], r2(L0["psi_b2"]),
      L0["mlp_W1"], r2(L0["mlp_b1"]), L0["mlp_W2"], r2(L0["mlp_b2"]),
      r2(L0["ln_g"]), r2(L0["ln_b"]),
      q0["W1"], r2(q0["b1"]), q0["W2"], r2(q0["b2"]))

    hn1, na1 = pl.pallas_call(
        _net_body,
        grid=(N_NET // BN,),
        in_specs=[_rows(BN, N_INST), _full((N_INST, EMB)), _rows(BN, EMB),
                  _full(L1["phi_W1"].shape), _full((1, EMB)),
                  _full(L1["phi_W2"].shape), _full((1, EMB))],
        out_specs=[_rows(BN, EMB), _rows(BN, 2 * EMB)],
        out_shape=[jax.ShapeDtypeStruct((N_NET, EMB), jnp.float32),
                   jax.ShapeDtypeStruct((N_NET, 2 * EMB), jnp.float32)],
    )(net_inst_adj, h_pre0, hn0, L1["phi_W1"], r2(L1["phi_b1"]),
      L1["phi_W2"], r2(L1["phi_b2"]))

    hd1, hs01 = _sc_spmv(bmd.reshape(N_INST * NGRP),
                         bms.reshape(N_INST * NGRP), na1)

    h_out1, h_in1 = pl.pallas_call(
        _tail1_body,
        grid=(8,),
        in_specs=[_rows(N_INST // 8, EMB), _rows(N_INST // 8, EMB),
                  _rows(N_INST // 8, EMB), _rows(N_INST // 8, NUM_VN),
                  _full((NUM_VN, EMB)),
                  _full(L1["psi_W1"].shape), _full((1, EMB)),
                  _full(L1["psi_W2"].shape), _full((1, EMB)),
                  _full(L1["mlp_W1"].shape), _full((1, 3 * EMB)),
                  _full(L1["mlp_W2"].shape), _full((1, EMB)),
                  _full((1, EMB)), _full((1, EMB))],
        out_specs=[_rows(N_INST // 8, EMB), _rows(N_INST // 8, EMB)],
        out_shape=[jax.ShapeDtypeStruct((N_INST, EMB), jnp.float32),
                   jax.ShapeDtypeStruct((N_INST, EMB), jnp.float32)],
    )(hd1, hs01, h_out0, oh, vn1,
      L1["psi_W1"], r2(L1["psi_b1"]), L1["psi_W2"], r2(L1["psi_b2"]),
      L1["mlp_W1"], r2(L1["mlp_b1"]), L1["mlp_W2"], r2(L1["mlp_b2"]),
      r2(L1["ln_g"]), r2(L1["ln_b"]))

    return jnp.concatenate([h_in0, h_in1, h_out1], axis=1)


# speedup vs baseline: 5.1132x; 5.1132x over previous
import functools

import jax
import jax.numpy as jnp
from jax.experimental import pallas as pl

N_INST = 8192
N_NET = 4096
EMB = 64
NUM_VN = 16

BR = 512
BN = 512


def _lrelu(v):
    return jnp.where(v >= 0, v, 0.1 * v)


def _dot(a, b):
    return jnp.dot(a, b, preferred_element_type=jnp.float32)


def _enc_inst_body(x_ref, w1_ref, b1_ref, w2_ref, b2_ref, o_ref):
    h = _lrelu(_dot(x_ref[...], w1_ref[...].T) + b1_ref[...])
    o_ref[...] = _lrelu(_dot(h, w2_ref[...].T) + b2_ref[...])


def _enc_net_body(x_ref, w1_ref, b1_ref, w2_ref, b2_ref,
                  p1_ref, pb1_ref, p2_ref, pb2_ref, hn_ref, na_ref):
    h = _lrelu(_dot(x_ref[...], w1_ref[...].T) + b1_ref[...])
    hn = _lrelu(_dot(h, w2_ref[...].T) + b2_ref[...])
    hn_ref[...] = hn
    t = jax.nn.relu(_dot(hn, p1_ref[...].T) + pb1_ref[...])
    na_ref[...] = _dot(t, p2_ref[...].T) + pb2_ref[...]


def _layer_body(drive_ref, sink_ref, na_ref, hb_ref, oh_ref, vnt_ref,
                psi1_ref, psib1_ref, psi2_ref, psib2_ref,
                m1_ref, mb1_ref, m2_ref, mb2_ref, g_ref, be_ref,
                q1_ref, qb1_ref, q2_ref, qb2_ref,
                ho_ref, hin_ref, hpre_ref, pool_ref, vn_next_ref, *, do_vn):
    i = pl.program_id(0)
    h_in = hb_ref[...] + _dot(oh_ref[...], vnt_ref[...])
    hin_ref[...] = h_in
    hd = _dot(drive_ref[...], na_ref[...])
    hs0 = _dot(sink_ref[...], na_ref[...])
    hs = _dot(jax.nn.relu(_dot(hs0, psi1_ref[...].T) + psib1_ref[...]),
              psi2_ref[...].T) + psib2_ref[...]
    hc = jnp.concatenate([h_in, hd, hs], axis=1)
    hm = jax.nn.relu(_dot(hc, m1_ref[...].T) + mb1_ref[...])
    ho = _dot(hm, m2_ref[...].T) + mb2_ref[...]
    hpre_ref[...] = ho
    mu = jnp.mean(ho, axis=-1, keepdims=True)
    var = jnp.mean((ho - mu) ** 2, axis=-1, keepdims=True)
    ho = (ho - mu) / jnp.sqrt(var + 1e-5) * g_ref[...] + be_ref[...]
    ho_ref[...] = _lrelu(ho) + h_in

    if do_vn:
        ones = jnp.ones((h_in.shape[0], EMB), jnp.float32)
        hp = jnp.concatenate([h_in, ones], axis=1)
        contrib = _dot(oh_ref[...].T, hp)

        @pl.when(i == 0)
        def _():
            pool_ref[...] = jnp.zeros_like(pool_ref)

        pool_ref[...] += contrib

        @pl.when(i == pl.num_programs(0) - 1)
        def _():
            pool = pool_ref[...]
            counts = jnp.maximum(pool[:, EMB:EMB + 1], 1.0)
            vn_in = pool[:, :EMB] / counts + vnt_ref[...]
            t = _lrelu(_dot(vn_in, q1_ref[...].T) + qb1_ref[...])
            vn_next_ref[...] = _lrelu(_dot(t, q2_ref[...].T) + qb2_ref[...])
    else:
        @pl.when(i == 0)
        def _():
            pool_ref[...] = jnp.zeros_like(pool_ref)
            vn_next_ref[...] = jnp.zeros_like(vn_next_ref)


def _net_body(adj_ref, ho_ref, hn_ref, p1_ref, pb1_ref, p2_ref, pb2_ref,
              hn1_ref, na1_ref):
    hn1 = _dot(adj_ref[...], ho_ref[...]) + hn_ref[...]
    hn1_ref[...] = hn1
    t = jax.nn.relu(_dot(hn1, p1_ref[...].T) + pb1_ref[...])
    na1_ref[...] = _dot(t, p2_ref[...].T) + pb2_ref[...]


def _full(shape):
    return pl.BlockSpec(shape, lambda i: tuple(0 for _ in shape))


def _rows(bs, width):
    return pl.BlockSpec((bs, width), lambda i: (i, 0))


def kernel(x, x_net, net_inst_adj, inst_net_adj_v_drive, inst_net_adj_v_sink,
           batch, num_vn, params):
    p = params
    r2 = lambda a: a.reshape(1, -1)
    oh = (batch[:, None] == jnp.arange(NUM_VN, dtype=batch.dtype)[None, :]
          ).astype(jnp.float32)
    vn0 = jnp.tile(p["vn_emb"], (NUM_VN, 1)) + 0.0 * num_vn

    h0 = pl.pallas_call(
        _enc_inst_body,
        grid=(8,),
        in_specs=[_rows(N_INST // 8, x.shape[1]),
                  _full(p["enc_W1"].shape), _full((1, 2 * EMB)),
                  _full(p["enc_W2"].shape), _full((1, EMB))],
        out_specs=_rows(N_INST // 8, EMB),
        out_shape=jax.ShapeDtypeStruct((N_INST, EMB), jnp.float32),
    )(x, p["enc_W1"], r2(p["enc_b1"]), p["enc_W2"], r2(p["enc_b2"]))

    L0, L1 = p["layers"][0], p["layers"][1]
    hn0, na0 = pl.pallas_call(
        _enc_net_body,
        grid=(4,),
        in_specs=[_rows(N_NET // 4, x_net.shape[1]),
                  _full(p["encnet_W1"].shape), _full((1, EMB)),
                  _full(p["encnet_W2"].shape), _full((1, EMB)),
                  _full(L0["phi_W1"].shape), _full((1, EMB)),
                  _full(L0["phi_W2"].shape), _full((1, EMB))],
        out_specs=[_rows(N_NET // 4, EMB), _rows(N_NET // 4, EMB)],
        out_shape=[jax.ShapeDtypeStruct((N_NET, EMB), jnp.float32),
                   jax.ShapeDtypeStruct((N_NET, EMB), jnp.float32)],
    )(x_net, p["encnet_W1"], r2(p["encnet_b1"]), p["encnet_W2"],
      r2(p["encnet_b2"]), L0["phi_W1"], r2(p["layers"][0]["phi_b1"]),
      L0["phi_W2"], r2(L0["phi_b2"]))

    def layer_call(L, q, drive, sink, na, h_base, vn_table, do_vn):
        grid = (N_INST // BR,)
        return pl.pallas_call(
            functools.partial(_layer_body, do_vn=do_vn),
            grid=grid,
            in_specs=[_rows(BR, N_NET), _rows(BR, N_NET), _full((N_NET, EMB)),
                      _rows(BR, EMB), _rows(BR, NUM_VN), _full((NUM_VN, EMB)),
                      _full(L["psi_W1"].shape), _full((1, EMB)),
                      _full(L["psi_W2"].shape), _full((1, EMB)),
                      _full(L["mlp_W1"].shape), _full((1, 3 * EMB)),
                      _full(L["mlp_W2"].shape), _full((1, EMB)),
                      _full((1, EMB)), _full((1, EMB)),
                      _full(q["W1"].shape), _full((1, 2 * EMB)),
                      _full(q["W2"].shape), _full((1, EMB))],
            out_specs=[_rows(BR, EMB), _rows(BR, EMB), _rows(BR, EMB),
                       _full((NUM_VN, 2 * EMB)), _full((NUM_VN, EMB))],
            out_shape=[jax.ShapeDtypeStruct((N_INST, EMB), jnp.float32),
                       jax.ShapeDtypeStruct((N_INST, EMB), jnp.float32),
                       jax.ShapeDtypeStruct((N_INST, EMB), jnp.float32),
                       jax.ShapeDtypeStruct((NUM_VN, 2 * EMB), jnp.float32),
                       jax.ShapeDtypeStruct((NUM_VN, EMB), jnp.float32)],
        )(drive, sink, na, h_base, oh, vn_table,
          L["psi_W1"], r2(L["psi_b1"]), L["psi_W2"], r2(L["psi_b2"]),
          L["mlp_W1"], r2(L["mlp_b1"]), L["mlp_W2"], r2(L["mlp_b2"]),
          r2(L["ln_g"]), r2(L["ln_b"]),
          q["W1"], r2(q["b1"]), q["W2"], r2(q["b2"]))

    q0 = p["vn_mlp"][0]
    h_out0, h_in0, h_pre0, _, vn1 = layer_call(
        L0, q0, inst_net_adj_v_drive, inst_net_adj_v_sink, na0, h0, vn0, True)

    hn1, na1 = pl.pallas_call(
        _net_body,
        grid=(N_NET // BN,),
        in_specs=[_rows(BN, N_INST), _full((N_INST, EMB)), _rows(BN, EMB),
                  _full(L1["phi_W1"].shape), _full((1, EMB)),
                  _full(L1["phi_W2"].shape), _full((1, EMB))],
        out_specs=[_rows(BN, EMB), _rows(BN, EMB)],
        out_shape=[jax.ShapeDtypeStruct((N_NET, EMB), jnp.float32),
                   jax.ShapeDtypeStruct((N_NET, EMB), jnp.float32)],
    )(net_inst_adj, h_pre0, hn0, L1["phi_W1"], r2(L1["phi_b1"]),
      L1["phi_W2"], r2(L1["phi_b2"]))

    h_out1, h_in1, _, _, _ = layer_call(
        L1, q0, inst_net_adj_v_drive, inst_net_adj_v_sink, na1, h_out0, vn1,
        False)

    return jnp.concatenate([h_in0, h_in1, h_out1], axis=1)


# speedup vs baseline: 5.2233x; 1.0215x over previous
import jax
import jax.numpy as jnp
from jax.experimental import pallas as pl

N_INST = 8192
N_NET = 4096
EMB = 64
NUM_VN = 16

BR = 512
BN = 512


def _lrelu(v):
    return jnp.where(v >= 0, v, 0.1 * v)


def _dot(a, b):
    return jnp.dot(a, b, preferred_element_type=jnp.float32)


def _enc(x, w1, b1, w2, b2):
    h = _lrelu(_dot(x, w1.T) + b1)
    return _lrelu(_dot(h, w2.T) + b2)


def _phi(hn, p1, pb1, p2, pb2):
    return _dot(jax.nn.relu(_dot(hn, p1.T) + pb1), p2.T) + pb2


def _tail_math(h_in, hd, hs0, psi1, psib1, psi2, psib2,
               m1, mb1, m2, mb2, g, be):
    hs = _dot(jax.nn.relu(_dot(hs0, psi1.T) + psib1), psi2.T) + psib2
    hc = jnp.concatenate([h_in, hd, hs], axis=1)
    hm = jax.nn.relu(_dot(hc, m1.T) + mb1)
    ho = _dot(hm, m2.T) + mb2
    mu = jnp.mean(ho, axis=-1, keepdims=True)
    var = jnp.mean((ho - mu) ** 2, axis=-1, keepdims=True)
    ln = (ho - mu) / jnp.sqrt(var + 1e-5) * g + be
    return ho, _lrelu(ln) + h_in


def _layer0_body(x_ref, xn_ref, drive_ref, sink_ref, oh_ref, vnt_ref,
                 e1_ref, eb1_ref, e2_ref, eb2_ref,
                 n1_ref, nb1_ref, n2_ref, nb2_ref,
                 p1_ref, pb1_ref, p2_ref, pb2_ref,
                 psi1_ref, psib1_ref, psi2_ref, psib2_ref,
                 m1_ref, mb1_ref, m2_ref, mb2_ref, g_ref, be_ref,
                 q1_ref, qb1_ref, q2_ref, qb2_ref,
                 ho_ref, hin_ref, hpre_ref, pool_ref, vn_next_ref):
    i = pl.program_id(0)
    h0 = _enc(x_ref[...], e1_ref[...], eb1_ref[...], e2_ref[...], eb2_ref[...])
    hn0 = _enc(xn_ref[...], n1_ref[...], nb1_ref[...], n2_ref[...],
               nb2_ref[...])
    na0 = _phi(hn0, p1_ref[...], pb1_ref[...], p2_ref[...], pb2_ref[...])
    h_in = h0 + _dot(oh_ref[...], vnt_ref[...])
    hin_ref[...] = h_in
    hd = _dot(drive_ref[...], na0)
    hs0 = _dot(sink_ref[...], na0)
    hpre, hout = _tail_math(
        h_in, hd, hs0, psi1_ref[...], psib1_ref[...], psi2_ref[...],
        psib2_ref[...], m1_ref[...], mb1_ref[...], m2_ref[...], mb2_ref[...],
        g_ref[...], be_ref[...])
    hpre_ref[...] = hpre
    ho_ref[...] = hout

    ones = jnp.ones((h_in.shape[0], EMB), jnp.float32)
    hp = jnp.concatenate([h_in, ones], axis=1)
    contrib = _dot(oh_ref[...].T, hp)

    @pl.when(i == 0)
    def _():
        pool_ref[...] = jnp.zeros_like(pool_ref)

    pool_ref[...] += contrib

    @pl.when(i == pl.num_programs(0) - 1)
    def _():
        pool = pool_ref[...]
        counts = jnp.maximum(pool[:, EMB:EMB + 1], 1.0)
        vn_in = pool[:, :EMB] / counts + vnt_ref[...]
        t = _lrelu(_dot(vn_in, q1_ref[...].T) + qb1_ref[...])
        vn_next_ref[...] = _lrelu(_dot(t, q2_ref[...].T) + qb2_ref[...])


def _net_body(adj_ref, hpre_ref, xn_ref,
              n1_ref, nb1_ref, n2_ref, nb2_ref,
              p1_ref, pb1_ref, p2_ref, pb2_ref, na1_ref):
    hn0 = _enc(xn_ref[...], n1_ref[...], nb1_ref[...], n2_ref[...],
               nb2_ref[...])
    hn1 = _dot(adj_ref[...], hpre_ref[...]) + hn0
    na1_ref[...] = _phi(hn1, p1_ref[...], pb1_ref[...], p2_ref[...],
                        pb2_ref[...])


def _layer1_body(drive_ref, sink_ref, na_ref, hb_ref, oh_ref, vnt_ref,
                 psi1_ref, psib1_ref, psi2_ref, psib2_ref,
                 m1_ref, mb1_ref, m2_ref, mb2_ref, g_ref, be_ref,
                 ho_ref, hin_ref):
    h_in = hb_ref[...] + _dot(oh_ref[...], vnt_ref[...])
    hin_ref[...] = h_in
    hd = _dot(drive_ref[...], na_ref[...])
    hs0 = _dot(sink_ref[...], na_ref[...])
    _, hout = _tail_math(
        h_in, hd, hs0, psi1_ref[...], psib1_ref[...], psi2_ref[...],
        psib2_ref[...], m1_ref[...], mb1_ref[...], m2_ref[...], mb2_ref[...],
        g_ref[...], be_ref[...])
    ho_ref[...] = hout


def _full(shape):
    return pl.BlockSpec(shape, lambda i: tuple(0 for _ in shape))


def _rows(bs, width):
    return pl.BlockSpec((bs, width), lambda i: (i, 0))


def kernel(x, x_net, net_inst_adj, inst_net_adj_v_drive, inst_net_adj_v_sink,
           batch, num_vn, params):
    p = params
    r2 = lambda a: a.reshape(1, -1)
    oh = (batch[:, None] == jnp.arange(NUM_VN, dtype=batch.dtype)[None, :]
          ).astype(jnp.float32)
    vn0 = jnp.tile(p["vn_emb"], (NUM_VN, 1)) + 0.0 * num_vn
    L0, L1 = p["layers"][0], p["layers"][1]
    q0 = p["vn_mlp"][0]

    h_out0, h_in0, h_pre0, _, vn1 = pl.pallas_call(
        _layer0_body,
        grid=(N_INST // BR,),
        in_specs=[_rows(BR, x.shape[1]), _full(x_net.shape),
                  _rows(BR, N_NET), _rows(BR, N_NET),
                  _rows(BR, NUM_VN), _full((NUM_VN, EMB)),
                  _full(p["enc_W1"].shape), _full((1, 2 * EMB)),
                  _full(p["enc_W2"].shape), _full((1, EMB)),
                  _full(p["encnet_W1"].shape), _full((1, EMB)),
                  _full(p["encnet_W2"].shape), _full((1, EMB)),
                  _full(L0["phi_W1"].shape), _full((1, EMB)),
                  _full(L0["phi_W2"].shape), _full((1, EMB)),
                  _full(L0["psi_W1"].shape), _full((1, EMB)),
                  _full(L0["psi_W2"].shape), _full((1, EMB)),
                  _full(L0["mlp_W1"].shape), _full((1, 3 * EMB)),
                  _full(L0["mlp_W2"].shape), _full((1, EMB)),
                  _full((1, EMB)), _full((1, EMB)),
                  _full(q0["W1"].shape), _full((1, 2 * EMB)),
                  _full(q0["W2"].shape), _full((1, EMB))],
        out_specs=[_rows(BR, EMB), _rows(BR, EMB), _rows(BR, EMB),
                   _full((NUM_VN, 2 * EMB)), _full((NUM_VN, EMB))],
        out_shape=[jax.ShapeDtypeStruct((N_INST, EMB), jnp.float32),
                   jax.ShapeDtypeStruct((N_INST, EMB), jnp.float32),
                   jax.ShapeDtypeStruct((N_INST, EMB), jnp.float32),
                   jax.ShapeDtypeStruct((NUM_VN, 2 * EMB), jnp.float32),
                   jax.ShapeDtypeStruct((NUM_VN, EMB), jnp.float32)],
    )(x, x_net, inst_net_adj_v_drive, inst_net_adj_v_sink, oh, vn0,
      p["enc_W1"], r2(p["enc_b1"]), p["enc_W2"], r2(p["enc_b2"]),
      p["encnet_W1"], r2(p["encnet_b1"]), p["encnet_W2"], r2(p["encnet_b2"]),
      L0["phi_W1"], r2(L0["phi_b1"]), L0["phi_W2"], r2(L0["phi_b2"]),
      L0["psi_W1"], r2(L0["psi_b1"]), L0["psi_W2"], r2(L0["psi_b2"]),
      L0["mlp_W1"], r2(L0["mlp_b1"]), L0["mlp_W2"], r2(L0["mlp_b2"]),
      r2(L0["ln_g"]), r2(L0["ln_b"]),
      q0["W1"], r2(q0["b1"]), q0["W2"], r2(q0["b2"]))

    na1 = pl.pallas_call(
        _net_body,
        grid=(N_NET // BN,),
        in_specs=[_rows(BN, N_INST), _full((N_INST, EMB)),
                  _rows(BN, x_net.shape[1]),
                  _full(p["encnet_W1"].shape), _full((1, EMB)),
                  _full(p["encnet_W2"].shape), _full((1, EMB)),
                  _full(L1["phi_W1"].shape), _full((1, EMB)),
                  _full(L1["phi_W2"].shape), _full((1, EMB))],
        out_specs=_rows(BN, EMB),
        out_shape=jax.ShapeDtypeStruct((N_NET, EMB), jnp.float32),
    )(net_inst_adj, h_pre0, x_net,
      p["encnet_W1"], r2(p["encnet_b1"]), p["encnet_W2"], r2(p["encnet_b2"]),
      L1["phi_W1"], r2(L1["phi_b1"]), L1["phi_W2"], r2(L1["phi_b2"]))

    h_out1, h_in1 = pl.pallas_call(
        _layer1_body,
        grid=(N_INST // BR,),
        in_specs=[_rows(BR, N_NET), _rows(BR, N_NET), _full((N_NET, EMB)),
                  _rows(BR, EMB), _rows(BR, NUM_VN), _full((NUM_VN, EMB)),
                  _full(L1["psi_W1"].shape), _full((1, EMB)),
                  _full(L1["psi_W2"].shape), _full((1, EMB)),
                  _full(L1["mlp_W1"].shape), _full((1, 3 * EMB)),
                  _full(L1["mlp_W2"].shape), _full((1, EMB)),
                  _full((1, EMB)), _full((1, EMB))],
        out_specs=[_rows(BR, EMB), _rows(BR, EMB)],
        out_shape=[jax.ShapeDtypeStruct((N_INST, EMB), jnp.float32),
                   jax.ShapeDtypeStruct((N_INST, EMB), jnp.float32)],
    )(inst_net_adj_v_drive, inst_net_adj_v_sink, na1, h_out0, oh, vn1,
      L1["psi_W1"], r2(L1["psi_b1"]), L1["psi_W2"], r2(L1["psi_b2"]),
      L1["mlp_W1"], r2(L1["mlp_b1"]), L1["mlp_W2"], r2(L1["mlp_b2"]),
      r2(L1["ln_g"]), r2(L1["ln_b"]))

    return jnp.concatenate([h_in0, h_in1, h_out1], axis=1)


# speedup vs baseline: 5.3631x; 1.0268x over previous
import jax
import jax.numpy as jnp
from jax.experimental import pallas as pl

N_INST = 8192
N_NET = 4096
EMB = 64
NUM_VN = 16

BR = 512
BN = 512


def _lrelu(v):
    return jnp.where(v >= 0, v, 0.1 * v)


def _dot(a, b):
    return jnp.dot(a, b, preferred_element_type=jnp.float32)


def _enc(x, w1, b1, w2, b2):
    h = _lrelu(_dot(x, w1.T) + b1)
    return _lrelu(_dot(h, w2.T) + b2)


def _phi(hn, p1, pb1, p2, pb2):
    return _dot(jax.nn.relu(_dot(hn, p1.T) + pb1), p2.T) + pb2


def _tail_math(h_in, hd, hs0, psi1, psib1, psi2, psib2,
               m1, mb1, m2, mb2, g, be):
    hs = _dot(jax.nn.relu(_dot(hs0, psi1.T) + psib1), psi2.T) + psib2
    hc = jnp.concatenate([h_in, hd, hs], axis=1)
    hm = jax.nn.relu(_dot(hc, m1.T) + mb1)
    ho = _dot(hm, m2.T) + mb2
    mu = jnp.mean(ho, axis=-1, keepdims=True)
    var = jnp.mean((ho - mu) ** 2, axis=-1, keepdims=True)
    ln = (ho - mu) / jnp.sqrt(var + 1e-5) * g + be
    return ho, _lrelu(ln) + h_in


def _layer0_body(x_ref, xn_ref, drive_ref, sink_ref, oh_ref, vnt_ref,
                 e1_ref, eb1_ref, e2_ref, eb2_ref,
                 n1_ref, nb1_ref, n2_ref, nb2_ref,
                 p1_ref, pb1_ref, p2_ref, pb2_ref,
                 psi1_ref, psib1_ref, psi2_ref, psib2_ref,
                 m1_ref, mb1_ref, m2_ref, mb2_ref, g_ref, be_ref,
                 q1_ref, qb1_ref, q2_ref, qb2_ref,
                 ho_ref, hin_ref, hpre_ref, di8_ref, si8_ref,
                 pool_ref, vn_next_ref):
    i = pl.program_id(0)
    h0 = _enc(x_ref[...], e1_ref[...], eb1_ref[...], e2_ref[...], eb2_ref[...])
    hn0 = _enc(xn_ref[...], n1_ref[...], nb1_ref[...], n2_ref[...],
               nb2_ref[...])
    na0 = _phi(hn0, p1_ref[...], pb1_ref[...], p2_ref[...], pb2_ref[...])
    h_in = h0 + _dot(oh_ref[...], vnt_ref[...])
    hin_ref[...] = h_in
    hd = _dot(drive_ref[...], na0)
    hs0 = _dot(sink_ref[...], na0)
    hpre, hout = _tail_math(
        h_in, hd, hs0, psi1_ref[...], psib1_ref[...], psi2_ref[...],
        psib2_ref[...], m1_ref[...], mb1_ref[...], m2_ref[...], mb2_ref[...],
        g_ref[...], be_ref[...])
    hpre_ref[...] = hpre
    ho_ref[...] = hout
    di8_ref[...] = drive_ref[...].astype(jnp.int8)
    si8_ref[...] = sink_ref[...].astype(jnp.int8)

    ones = jnp.ones((h_in.shape[0], EMB), jnp.float32)
    hp = jnp.concatenate([h_in, ones], axis=1)
    contrib = _dot(oh_ref[...].T, hp)

    @pl.when(i == 0)
    def _():
        pool_ref[...] = jnp.zeros_like(pool_ref)

    pool_ref[...] += contrib

    @pl.when(i == pl.num_programs(0) - 1)
    def _():
        pool = pool_ref[...]
        counts = jnp.maximum(pool[:, EMB:EMB + 1], 1.0)
        vn_in = pool[:, :EMB] / counts + vnt_ref[...]
        t = _lrelu(_dot(vn_in, q1_ref[...].T) + qb1_ref[...])
        vn_next_ref[...] = _lrelu(_dot(t, q2_ref[...].T) + qb2_ref[...])


def _net_body(adj_ref, hpre_ref, xn_ref,
              n1_ref, nb1_ref, n2_ref, nb2_ref,
              p1_ref, pb1_ref, p2_ref, pb2_ref, na1_ref):
    hn0 = _enc(xn_ref[...], n1_ref[...], nb1_ref[...], n2_ref[...],
               nb2_ref[...])
    hn1 = _dot(adj_ref[...], hpre_ref[...]) + hn0
    na1_ref[...] = _phi(hn1, p1_ref[...], pb1_ref[...], p2_ref[...],
                        pb2_ref[...])


def _layer1_body(drive_ref, sink_ref, na_ref, hb_ref, oh_ref, vnt_ref,
                 psi1_ref, psib1_ref, psi2_ref, psib2_ref,
                 m1_ref, mb1_ref, m2_ref, mb2_ref, g_ref, be_ref,
                 ho_ref, hin_ref):
    h_in = hb_ref[...] + _dot(oh_ref[...], vnt_ref[...])
    hin_ref[...] = h_in
    na_bf = na_ref[...].astype(jnp.bfloat16)
    hd = _dot(drive_ref[...].astype(jnp.bfloat16), na_bf)
    hs0 = _dot(sink_ref[...].astype(jnp.bfloat16), na_bf)
    _, hout = _tail_math(
        h_in, hd, hs0, psi1_ref[...], psib1_ref[...], psi2_ref[...],
        psib2_ref[...], m1_ref[...], mb1_ref[...], m2_ref[...], mb2_ref[...],
        g_ref[...], be_ref[...])
    ho_ref[...] = hout


def _full(shape):
    return pl.BlockSpec(shape, lambda i: tuple(0 for _ in shape))


def _rows(bs, width):
    return pl.BlockSpec((bs, width), lambda i: (i, 0))


def kernel(x, x_net, net_inst_adj, inst_net_adj_v_drive, inst_net_adj_v_sink,
           batch, num_vn, params):
    p = params
    r2 = lambda a: a.reshape(1, -1)
    oh = (batch[:, None] == jnp.arange(NUM_VN, dtype=batch.dtype)[None, :]
          ).astype(jnp.float32)
    vn0 = jnp.tile(p["vn_emb"], (NUM_VN, 1)) + 0.0 * num_vn
    L0, L1 = p["layers"][0], p["layers"][1]
    q0 = p["vn_mlp"][0]

    h_out0, h_in0, h_pre0, drive_i8, sink_i8, _, vn1 = pl.pallas_call(
        _layer0_body,
        grid=(N_INST // BR,),
        in_specs=[_rows(BR, x.shape[1]), _full(x_net.shape),
                  _rows(BR, N_NET), _rows(BR, N_NET),
                  _rows(BR, NUM_VN), _full((NUM_VN, EMB)),
                  _full(p["enc_W1"].shape), _full((1, 2 * EMB)),
                  _full(p["enc_W2"].shape), _full((1, EMB)),
                  _full(p["encnet_W1"].shape), _full((1, EMB)),
                  _full(p["encnet_W2"].shape), _full((1, EMB)),
                  _full(L0["phi_W1"].shape), _full((1, EMB)),
                  _full(L0["phi_W2"].shape), _full((1, EMB)),
                  _full(L0["psi_W1"].shape), _full((1, EMB)),
                  _full(L0["psi_W2"].shape), _full((1, EMB)),
                  _full(L0["mlp_W1"].shape), _full((1, 3 * EMB)),
                  _full(L0["mlp_W2"].shape), _full((1, EMB)),
                  _full((1, EMB)), _full((1, EMB)),
                  _full(q0["W1"].shape), _full((1, 2 * EMB)),
                  _full(q0["W2"].shape), _full((1, EMB))],
        out_specs=[_rows(BR, EMB), _rows(BR, EMB), _rows(BR, EMB),
                   _rows(BR, N_NET), _rows(BR, N_NET),
                   _full((NUM_VN, 2 * EMB)), _full((NUM_VN, EMB))],
        out_shape=[jax.ShapeDtypeStruct((N_INST, EMB), jnp.float32),
                   jax.ShapeDtypeStruct((N_INST, EMB), jnp.float32),
                   jax.ShapeDtypeStruct((N_INST, EMB), jnp.float32),
                   jax.ShapeDtypeStruct((N_INST, N_NET), jnp.int8),
                   jax.ShapeDtypeStruct((N_INST, N_NET), jnp.int8),
                   jax.ShapeDtypeStruct((NUM_VN, 2 * EMB), jnp.float32),
                   jax.ShapeDtypeStruct((NUM_VN, EMB), jnp.float32)],
    )(x, x_net, inst_net_adj_v_drive, inst_net_adj_v_sink, oh, vn0,
      p["enc_W1"], r2(p["enc_b1"]), p["enc_W2"], r2(p["enc_b2"]),
      p["encnet_W1"], r2(p["encnet_b1"]), p["encnet_W2"], r2(p["encnet_b2"]),
      L0["phi_W1"], r2(L0["phi_b1"]), L0["phi_W2"], r2(L0["phi_b2"]),
      L0["psi_W1"], r2(L0["psi_b1"]), L0["psi_W2"], r2(L0["psi_b2"]),
      L0["mlp_W1"], r2(L0["mlp_b1"]), L0["mlp_W2"], r2(L0["mlp_b2"]),
      r2(L0["ln_g"]), r2(L0["ln_b"]),
      q0["W1"], r2(q0["b1"]), q0["W2"], r2(q0["b2"]))

    na1 = pl.pallas_call(
        _net_body,
        grid=(N_NET // BN,),
        in_specs=[_rows(BN, N_INST), _full((N_INST, EMB)),
                  _rows(BN, x_net.shape[1]),
                  _full(p["encnet_W1"].shape), _full((1, EMB)),
                  _full(p["encnet_W2"].shape), _full((1, EMB)),
                  _full(L1["phi_W1"].shape), _full((1, EMB)),
                  _full(L1["phi_W2"].shape), _full((1, EMB))],
        out_specs=_rows(BN, EMB),
        out_shape=jax.ShapeDtypeStruct((N_NET, EMB), jnp.float32),
    )(net_inst_adj, h_pre0, x_net,
      p["encnet_W1"], r2(p["encnet_b1"]), p["encnet_W2"], r2(p["encnet_b2"]),
      L1["phi_W1"], r2(L1["phi_b1"]), L1["phi_W2"], r2(L1["phi_b2"]))

    h_out1, h_in1 = pl.pallas_call(
        _layer1_body,
        grid=(N_INST // BR,),
        in_specs=[_rows(BR, N_NET), _rows(BR, N_NET), _full((N_NET, EMB)),
                  _rows(BR, EMB), _rows(BR, NUM_VN), _full((NUM_VN, EMB)),
                  _full(L1["psi_W1"].shape), _full((1, EMB)),
                  _full(L1["psi_W2"].shape), _full((1, EMB)),
                  _full(L1["mlp_W1"].shape), _full((1, 3 * EMB)),
                  _full(L1["mlp_W2"].shape), _full((1, EMB)),
                  _full((1, EMB)), _full((1, EMB))],
        out_specs=[_rows(BR, EMB), _rows(BR, EMB)],
        out_shape=[jax.ShapeDtypeStruct((N_INST, EMB), jnp.float32),
                   jax.ShapeDtypeStruct((N_INST, EMB), jnp.float32)],
    )(drive_i8, sink_i8, na1, h_out0, oh, vn1,
      L1["psi_W1"], r2(L1["psi_b1"]), L1["psi_W2"], r2(L1["psi_b2"]),
      L1["mlp_W1"], r2(L1["mlp_b1"]), L1["mlp_W2"], r2(L1["mlp_b2"]),
      r2(L1["ln_g"]), r2(L1["ln_b"]))

    return jnp.concatenate([h_in0, h_in1, h_out1], axis=1)


# speedup vs baseline: 5.4701x; 1.0199x over previous
import jax
import jax.numpy as jnp
from jax.experimental import pallas as pl

N_INST = 8192
N_NET = 4096
EMB = 64
NUM_VN = 16

BR = 512
BN = 512
BC = 1024


def _lrelu(v):
    return jnp.where(v >= 0, v, 0.1 * v)


def _dot(a, b):
    return jnp.dot(a, b, preferred_element_type=jnp.float32)


def _enc(x, w1, b1, w2, b2):
    h = _lrelu(_dot(x, w1.T) + b1)
    return _lrelu(_dot(h, w2.T) + b2)


def _phi(hn, p1, pb1, p2, pb2):
    return _dot(jax.nn.relu(_dot(hn, p1.T) + pb1), p2.T) + pb2


def _tail_math(h_in, hd, hs0, psi1, psib1, psi2, psib2,
               m1, mb1, m2, mb2, g, be):
    hs = _dot(jax.nn.relu(_dot(hs0, psi1.T) + psib1), psi2.T) + psib2
    hc = jnp.concatenate([h_in, hd, hs], axis=1)
    hm = jax.nn.relu(_dot(hc, m1.T) + mb1)
    ho = _dot(hm, m2.T) + mb2
    mu = jnp.mean(ho, axis=-1, keepdims=True)
    var = jnp.mean((ho - mu) ** 2, axis=-1, keepdims=True)
    ln = (ho - mu) / jnp.sqrt(var + 1e-5) * g + be
    return ho, _lrelu(ln) + h_in


def _layer0_body(x_ref, xn_ref, drive_ref, sink_ref, oh_ref, vnt_ref,
                 e1_ref, eb1_ref, e2_ref, eb2_ref,
                 n1_ref, nb1_ref, n2_ref, nb2_ref,
                 p1_ref, pb1_ref, p2_ref, pb2_ref,
                 psi1_ref, psib1_ref, psi2_ref, psib2_ref,
                 m1_ref, mb1_ref, m2_ref, mb2_ref, g_ref, be_ref,
                 q1_ref, qb1_ref, q2_ref, qb2_ref,
                 ho_ref, hin_ref, hpre_ref, di8_ref, si8_ref,
                 pool_ref, vn_next_ref):
    i = pl.program_id(0)
    h0 = _enc(x_ref[...], e1_ref[...], eb1_ref[...], e2_ref[...], eb2_ref[...])
    hn0 = _enc(xn_ref[...], n1_ref[...], nb1_ref[...], n2_ref[...],
               nb2_ref[...])
    na0 = _phi(hn0, p1_ref[...], pb1_ref[...], p2_ref[...], pb2_ref[...])
    h_in = h0 + _dot(oh_ref[...], vnt_ref[...])
    hin_ref[...] = h_in
    hd = _dot(drive_ref[...], na0)
    hs0 = _dot(sink_ref[...], na0)
    hpre, hout = _tail_math(
        h_in, hd, hs0, psi1_ref[...], psib1_ref[...], psi2_ref[...],
        psib2_ref[...], m1_ref[...], mb1_ref[...], m2_ref[...], mb2_ref[...],
        g_ref[...], be_ref[...])
    hpre_ref[...] = hpre
    ho_ref[...] = hout
    di8_ref[...] = drive_ref[...].astype(jnp.int8)
    si8_ref[...] = sink_ref[...].astype(jnp.int8)

    ones = jnp.ones((h_in.shape[0], EMB), jnp.float32)
    hp = jnp.concatenate([h_in, ones], axis=1)
    contrib = _dot(oh_ref[...].T, hp)

    @pl.when(i == 0)
    def _():
        pool_ref[...] = jnp.zeros_like(pool_ref)

    pool_ref[...] += contrib

    @pl.when(i == pl.num_programs(0) - 1)
    def _():
        pool = pool_ref[...]
        counts = jnp.maximum(pool[:, EMB:EMB + 1], 1.0)
        vn_in = pool[:, :EMB] / counts + vnt_ref[...]
        t = _lrelu(_dot(vn_in, q1_ref[...].T) + qb1_ref[...])
        vn_next_ref[...] = _lrelu(_dot(t, q2_ref[...].T) + qb2_ref[...])


def _net_body(adj_ref, hpre_ref, xn_ref,
              n1_ref, nb1_ref, n2_ref, nb2_ref,
              p1_ref, pb1_ref, p2_ref, pb2_ref, na1_ref):
    hn0 = _enc(xn_ref[...], n1_ref[...], nb1_ref[...], n2_ref[...],
               nb2_ref[...])
    hn1 = _dot(adj_ref[...], hpre_ref[...]) + hn0
    na1_ref[...] = _phi(hn1, p1_ref[...], pb1_ref[...], p2_ref[...],
                        pb2_ref[...])


def _layer1_body(drive_ref, sink_ref, na_ref, hb_ref, oh_ref, vnt_ref,
                 psi1_ref, psib1_ref, psi2_ref, psib2_ref,
                 m1_ref, mb1_ref, m2_ref, mb2_ref, g_ref, be_ref,
                 ho_ref, hin_ref):
    h_in = hb_ref[...] + _dot(oh_ref[...], vnt_ref[...])
    hin_ref[...] = h_in
    na_bf = na_ref[...].astype(jnp.bfloat16)
    hd = _dot(drive_ref[...].astype(jnp.bfloat16), na_bf)
    hs0 = _dot(sink_ref[...].astype(jnp.bfloat16), na_bf)
    _, hout = _tail_math(
        h_in, hd, hs0, psi1_ref[...], psib1_ref[...], psi2_ref[...],
        psib2_ref[...], m1_ref[...], mb1_ref[...], m2_ref[...], mb2_ref[...],
        g_ref[...], be_ref[...])
    ho_ref[...] = hout


def _full(shape):
    return pl.BlockSpec(shape, lambda i: tuple(0 for _ in shape))


def _rows(bs, width):
    return pl.BlockSpec((bs, width), lambda i: (i, 0))


def kernel(x, x_net, net_inst_adj, inst_net_adj_v_drive, inst_net_adj_v_sink,
           batch, num_vn, params):
    p = params
    r2 = lambda a: a.reshape(1, -1)
    oh = (batch[:, None] == jnp.arange(NUM_VN, dtype=batch.dtype)[None, :]
          ).astype(jnp.float32)
    vn0 = jnp.tile(p["vn_emb"], (NUM_VN, 1)) + 0.0 * num_vn
    L0, L1 = p["layers"][0], p["layers"][1]
    q0 = p["vn_mlp"][0]

    h_out0, h_in0, h_pre0, drive_i8, sink_i8, _, vn1 = pl.pallas_call(
        _layer0_body,
        grid=(N_INST // BR,),
        in_specs=[_rows(BR, x.shape[1]), _full(x_net.shape),
                  _rows(BR, N_NET), _rows(BR, N_NET),
                  _rows(BR, NUM_VN), _full((NUM_VN, EMB)),
                  _full(p["enc_W1"].shape), _full((1, 2 * EMB)),
                  _full(p["enc_W2"].shape), _full((1, EMB)),
                  _full(p["encnet_W1"].shape), _full((1, EMB)),
                  _full(p["encnet_W2"].shape), _full((1, EMB)),
                  _full(L0["phi_W1"].shape), _full((1, EMB)),
                  _full(L0["phi_W2"].shape), _full((1, EMB)),
                  _full(L0["psi_W1"].shape), _full((1, EMB)),
                  _full(L0["psi_W2"].shape), _full((1, EMB)),
                  _full(L0["mlp_W1"].shape), _full((1, 3 * EMB)),
                  _full(L0["mlp_W2"].shape), _full((1, EMB)),
                  _full((1, EMB)), _full((1, EMB)),
                  _full(q0["W1"].shape), _full((1, 2 * EMB)),
                  _full(q0["W2"].shape), _full((1, EMB))],
        out_specs=[_rows(BR, EMB), _rows(BR, EMB), _rows(BR, EMB),
                   _rows(BR, N_NET), _rows(BR, N_NET),
                   _full((NUM_VN, 2 * EMB)), _full((NUM_VN, EMB))],
        out_shape=[jax.ShapeDtypeStruct((N_INST, EMB), jnp.float32),
                   jax.ShapeDtypeStruct((N_INST, EMB), jnp.float32),
                   jax.ShapeDtypeStruct((N_INST, EMB), jnp.float32),
                   jax.ShapeDtypeStruct((N_INST, N_NET), jnp.int8),
                   jax.ShapeDtypeStruct((N_INST, N_NET), jnp.int8),
                   jax.ShapeDtypeStruct((NUM_VN, 2 * EMB), jnp.float32),
                   jax.ShapeDtypeStruct((NUM_VN, EMB), jnp.float32)],
    )(x, x_net, inst_net_adj_v_drive, inst_net_adj_v_sink, oh, vn0,
      p["enc_W1"], r2(p["enc_b1"]), p["enc_W2"], r2(p["enc_b2"]),
      p["encnet_W1"], r2(p["encnet_b1"]), p["encnet_W2"], r2(p["encnet_b2"]),
      L0["phi_W1"], r2(L0["phi_b1"]), L0["phi_W2"], r2(L0["phi_b2"]),
      L0["psi_W1"], r2(L0["psi_b1"]), L0["psi_W2"], r2(L0["psi_b2"]),
      L0["mlp_W1"], r2(L0["mlp_b1"]), L0["mlp_W2"], r2(L0["mlp_b2"]),
      r2(L0["ln_g"]), r2(L0["ln_b"]),
      q0["W1"], r2(q0["b1"]), q0["W2"], r2(q0["b2"]))

    na1 = pl.pallas_call(
        _net_body,
        grid=(N_NET // BN,),
        in_specs=[_rows(BN, N_INST), _full((N_INST, EMB)),
                  _rows(BN, x_net.shape[1]),
                  _full(p["encnet_W1"].shape), _full((1, EMB)),
                  _full(p["encnet_W2"].shape), _full((1, EMB)),
                  _full(L1["phi_W1"].shape), _full((1, EMB)),
                  _full(L1["phi_W2"].shape), _full((1, EMB))],
        out_specs=_rows(BN, EMB),
        out_shape=jax.ShapeDtypeStruct((N_NET, EMB), jnp.float32),
    )(net_inst_adj, h_pre0, x_net,
      p["encnet_W1"], r2(p["encnet_b1"]), p["encnet_W2"], r2(p["encnet_b2"]),
      L1["phi_W1"], r2(L1["phi_b1"]), L1["phi_W2"], r2(L1["phi_b2"]))

    h_out1, h_in1 = pl.pallas_call(
        _layer1_body,
        grid=(N_INST // BC,),
        in_specs=[_rows(BC, N_NET), _rows(BC, N_NET), _full((N_NET, EMB)),
                  _rows(BC, EMB), _rows(BC, NUM_VN), _full((NUM_VN, EMB)),
                  _full(L1["psi_W1"].shape), _full((1, EMB)),
                  _full(L1["psi_W2"].shape), _full((1, EMB)),
                  _full(L1["mlp_W1"].shape), _full((1, 3 * EMB)),
                  _full(L1["mlp_W2"].shape), _full((1, EMB)),
                  _full((1, EMB)), _full((1, EMB))],
        out_specs=[_rows(BC, EMB), _rows(BC, EMB)],
        out_shape=[jax.ShapeDtypeStruct((N_INST, EMB), jnp.float32),
                   jax.ShapeDtypeStruct((N_INST, EMB), jnp.float32)],
    )(drive_i8, sink_i8, na1, h_out0, oh, vn1,
      L1["psi_W1"], r2(L1["psi_b1"]), L1["psi_W2"], r2(L1["psi_b2"]),
      L1["mlp_W1"], r2(L1["mlp_b1"]), L1["mlp_W2"], r2(L1["mlp_b2"]),
      r2(L1["ln_g"]), r2(L1["ln_b"]))

    return jnp.concatenate([h_in0, h_in1, h_out1], axis=1)
